# baseline (speedup 1.0000x reference)
import functools

import jax
import jax.numpy as jnp
from jax import lax
from jax.experimental import pallas as pl
from jax.experimental.pallas import tpu as pltpu
from jax.experimental.pallas import tpu_sc as plsc

_NC = 2
_NS = 16
_NW = _NC * _NS
_LANES = 16
_NBUF = 3
_DP = 128


def _make_sc_pool(B, L, V, D):
    bpw = B // _NW
    na = 128
    nb = L - na
    nchunk = D // _LANES

    mesh = plsc.VectorSubcoreMesh(
        core_axis_name="c", subcore_axis_name="s",
        num_cores=_NC, num_subcores=_NS)

    @functools.partial(
        pl.kernel,
        out_type=jax.ShapeDtypeStruct((B, D), jnp.float32),
        mesh=mesh,
        scratch_types=[
            pltpu.VMEM((bpw, L), jnp.int32),
            pltpu.VMEM((_NBUF, L, _DP), jnp.float32),
            pltpu.VMEM((bpw, D), jnp.float32),
            pltpu.SemaphoreType.DMA,
            pltpu.SemaphoreType.DMA,
            pltpu.SemaphoreType.DMA,
        ],
    )
    def sc_pool(idx_hbm, table_hbm, out_hbm, idx_v, rows_v, pooled_v,
                sem0, sem1, sem2):
        sems = (sem0, sem1, sem2)
        wid = lax.axis_index("s") * _NC + lax.axis_index("c")
        base = wid * bpw
        pltpu.sync_copy(idx_hbm.at[pl.ds(base, bpw)], idx_v)

        def issue(r, buf):
            pltpu.async_copy(table_hbm.at[idx_v.at[r, pl.ds(0, na)]],
                             rows_v.at[buf, pl.ds(0, na)], sems[buf])
            pltpu.async_copy(table_hbm.at[idx_v.at[r, pl.ds(na, nb)]],
                             rows_v.at[buf, pl.ds(na, nb)], sems[buf])

        def wait(buf):
            pltpu.make_async_copy(table_hbm.at[pl.ds(0, L)],
                                  rows_v.at[buf], sems[buf]).wait()

        def accum_store(r, buf):
            def body(i, accs):
                return tuple(
                    a + rows_v[buf, i, pl.ds(c * _LANES, _LANES)]
                    for c, a in enumerate(accs))
            zero = jnp.zeros((_LANES,), jnp.float32)
            accs = lax.fori_loop(0, L, body, (zero,) * nchunk)
            for c in range(nchunk):
                pooled_v[r, pl.ds(c * _LANES, _LANES)] = accs[c]

        for buf in range(_NBUF):
            issue(buf, buf)

        def outer(k, carry):
            r0 = k * _NBUF
            for buf in range(_NBUF):
                wait(buf)
                accum_store(r0 + buf, buf)
                issue(r0 + buf + _NBUF, buf)
            return carry

        n_full = bpw // _NBUF - 1
        lax.fori_loop(0, n_full, outer, 0)
        r0 = n_full * _NBUF
        nissued = r0 + _NBUF
        for j in range(bpw - r0):
            r = r0 + j
            wait(r % _NBUF)
            accum_store(r, r % _NBUF)
            if nissued < bpw:
                issue(nissued, nissued % _NBUF)
                nissued += 1

        pltpu.sync_copy(pooled_v, out_hbm.at[pl.ds(base, bpw)])

    return sc_pool


def _transpose_body(x_ref, o_ref):
    xt = x_ref[...].T
    o_ref[...] = jnp.concatenate(
        [xt, jnp.zeros_like(xt)], axis=1)


def _widen_table(tableT, tile_v=512):
    D, V = tableT.shape
    grid = (pl.cdiv(V, tile_v),)
    return pl.pallas_call(
        _transpose_body,
        grid=grid,
        in_specs=[pl.BlockSpec((D, tile_v), lambda i: (0, i))],
        out_specs=pl.BlockSpec((tile_v, _DP), lambda i: (i, 0)),
        out_shape=jax.ShapeDtypeStruct((V, _DP), jnp.float32),
    )(tableT)


def _mlp_body(x_ref, tl_ref, w1_ref, b1_ref, w2_ref, b2_ref, o_ref):
    x = x_ref[...] / tl_ref[...]
    h = lax.dot_general(x, w1_ref[...], (((1,), (1,)), ((), ())),
                        preferred_element_type=jnp.float32)
    h = jnp.maximum(h + b1_ref[...], 0.0)
    o = lax.dot_general(h, w2_ref[...], (((1,), (1,)), ((), ())),
                        preferred_element_type=jnp.float32)
    o_ref[...] = o + b2_ref[...]


def _mlp(pooled, text_len, W1, b1, W2, b2, tile_b=512):
    B, D = pooled.shape
    H = W1.shape[0]
    C = W2.shape[0]
    grid = (B // tile_b,)
    return pl.pallas_call(
        _mlp_body,
        grid=grid,
        in_specs=[
            pl.BlockSpec((tile_b, D), lambda i: (i, 0)),
            pl.BlockSpec((tile_b, 1), lambda i: (i, 0)),
            pl.BlockSpec((H, D), lambda i: (0, 0)),
            pl.BlockSpec((1, H), lambda i: (0, 0)),
            pl.BlockSpec((C, H), lambda i: (0, 0)),
            pl.BlockSpec((1, C), lambda i: (0, 0)),
        ],
        out_specs=pl.BlockSpec((tile_b, C), lambda i: (i, 0)),
        out_shape=jax.ShapeDtypeStruct((B, C), jnp.float32),
    )(pooled, text_len.reshape(B, 1), W1, b1.reshape(1, H), W2,
      b2.reshape(1, C))


def kernel(input_text, text_len, table, W1, b1, W2, b2):
    B, L = input_text.shape
    V, D = table.shape
    tab128 = _widen_table(table.T)
    pooled = _make_sc_pool(B, L, V, D)(input_text, tab128)
    return _mlp(pooled, text_len, W1, b1, W2, b2)


# speedup vs baseline: 2.9926x; 2.9926x over previous
import functools

import jax
import jax.numpy as jnp
from jax import lax
from jax.experimental import pallas as pl
from jax.experimental.pallas import tpu as pltpu
from jax.experimental.pallas import tpu_sc as plsc

_NC = 2
_NS = 16
_NW = _NC * _NS
_LANES = 16
_NBUF = 3
_DP = 128


def _make_sc_pool(B, L, V, D):
    bpw = B // _NW
    na = 128
    nb = L - na
    nchunk = D // _LANES

    mesh = plsc.VectorSubcoreMesh(
        core_axis_name="c", subcore_axis_name="s",
        num_cores=_NC, num_subcores=_NS)

    @functools.partial(
        pl.kernel,
        out_type=jax.ShapeDtypeStruct((B, D), jnp.float32),
        mesh=mesh,
        scratch_types=[
            pltpu.VMEM((bpw, L), jnp.int32),
            pltpu.VMEM((_NBUF, L, _DP), jnp.float32),
            pltpu.VMEM((bpw, D), jnp.float32),
            pltpu.SemaphoreType.DMA,
            pltpu.SemaphoreType.DMA,
            pltpu.SemaphoreType.DMA,
        ],
    )
    def sc_pool(idx_hbm, table_hbm, out_hbm, idx_v, rows_v, pooled_v,
                sem0, sem1, sem2):
        sems = (sem0, sem1, sem2)
        wid = lax.axis_index("s") * _NC + lax.axis_index("c")
        base = wid * bpw
        pltpu.sync_copy(idx_hbm.at[pl.ds(base, bpw)], idx_v)

        def issue(r, buf):
            pltpu.async_copy(table_hbm.at[idx_v.at[r, pl.ds(0, na)]],
                             rows_v.at[buf, pl.ds(0, na)], sems[buf])
            pltpu.async_copy(table_hbm.at[idx_v.at[r, pl.ds(na, nb)]],
                             rows_v.at[buf, pl.ds(na, nb)], sems[buf])

        def wait(buf):
            pltpu.make_async_copy(table_hbm.at[pl.ds(0, L)],
                                  rows_v.at[buf], sems[buf]).wait()

        def accum_store(r, buf):
            def body(i, accs):
                return tuple(
                    a + rows_v[buf, i, pl.ds(c * _LANES, _LANES)]
                    for c, a in enumerate(accs))
            zero = jnp.zeros((_LANES,), jnp.float32)
            accs = lax.fori_loop(0, L, body, (zero,) * nchunk)
            for c in range(nchunk):
                pooled_v[r, pl.ds(c * _LANES, _LANES)] = accs[c]

        for buf in range(_NBUF):
            issue(buf, buf)

        def outer(k, carry):
            r0 = k * _NBUF
            for buf in range(_NBUF):
                wait(buf)
                accum_store(r0 + buf, buf)
                issue(r0 + buf + _NBUF, buf)
            return carry

        n_full = bpw // _NBUF - 1
        lax.fori_loop(0, n_full, outer, 0)
        r0 = n_full * _NBUF
        nissued = r0 + _NBUF
        for j in range(bpw - r0):
            r = r0 + j
            wait(r % _NBUF)
            accum_store(r, r % _NBUF)
            if nissued < bpw:
                issue(nissued, nissued % _NBUF)
                nissued += 1

        pltpu.sync_copy(pooled_v, out_hbm.at[pl.ds(base, bpw)])

    return sc_pool


def _transpose_body(x_ref, o_ref):
    xt = x_ref[...].T
    o_ref[...] = jnp.concatenate(
        [xt, jnp.zeros_like(xt)], axis=1)


def _widen_table(tableT, tile_v=8192):
    D, V = tableT.shape
    grid = (pl.cdiv(V, tile_v),)
    return pl.pallas_call(
        _transpose_body,
        grid=grid,
        in_specs=[pl.BlockSpec((D, tile_v), lambda i: (0, i))],
        out_specs=pl.BlockSpec((tile_v, _DP), lambda i: (i, 0)),
        out_shape=jax.ShapeDtypeStruct((V, _DP), jnp.float32),
    )(tableT)


def _mlp_body(x_ref, tl_ref, w1_ref, b1_ref, w2_ref, b2_ref, o_ref):
    x = x_ref[...] / tl_ref[...]
    h = lax.dot_general(x, w1_ref[...], (((1,), (1,)), ((), ())),
                        preferred_element_type=jnp.float32)
    h = jnp.maximum(h + b1_ref[...], 0.0)
    o = lax.dot_general(h, w2_ref[...], (((1,), (1,)), ((), ())),
                        preferred_element_type=jnp.float32)
    o_ref[...] = o + b2_ref[...]


def _mlp(pooled, text_len, W1, b1, W2, b2, tile_b=512):
    B, D = pooled.shape
    H = W1.shape[0]
    C = W2.shape[0]
    grid = (B // tile_b,)
    return pl.pallas_call(
        _mlp_body,
        grid=grid,
        in_specs=[
            pl.BlockSpec((tile_b, D), lambda i: (i, 0)),
            pl.BlockSpec((tile_b, 1), lambda i: (i, 0)),
            pl.BlockSpec((H, D), lambda i: (0, 0)),
            pl.BlockSpec((1, H), lambda i: (0, 0)),
            pl.BlockSpec((C, H), lambda i: (0, 0)),
            pl.BlockSpec((1, C), lambda i: (0, 0)),
        ],
        out_specs=pl.BlockSpec((tile_b, C), lambda i: (i, 0)),
        out_shape=jax.ShapeDtypeStruct((B, C), jnp.float32),
    )(pooled, text_len.reshape(B, 1), W1, b1.reshape(1, H), W2,
      b2.reshape(1, C))


def kernel(input_text, text_len, table, W1, b1, W2, b2):
    B, L = input_text.shape
    V, D = table.shape
    tab128 = _widen_table(table.T)
    pooled = _make_sc_pool(B, L, V, D)(input_text, tab128)
    return _mlp(pooled, text_len, W1, b1, W2, b2)
